# baseline (speedup 1.0000x reference)
import functools

import numpy as np

import jax
import jax.numpy as jnp
from jax import lax
from jax.experimental import pallas as pl
from jax.experimental.pallas import tpu as pltpu
from jax.experimental.pallas import tpu_sc as plsc

D = 200
S = 25
K = 3
RW = 80
B = 4096
NC, NS, L = 2, 16, 16
NW = NC * NS
PER_W = B // NW
COLS = tuple(c * L for c in range(12)) + (D - L,)

_SPOS = np.zeros(RW, np.int32)
for _j in range(1, 1 + S * K):
    _k = 1 + (_j - 1) // S
    _s = (_j - 1) % S
    _SPOS[_j] = _s * (K + 1) + _k


def _sc_scores(emb, wpad, spos, uind, iind, uwalks, iwalks):
    mesh = plsc.VectorSubcoreMesh(core_axis_name="c", subcore_axis_name="s")

    @functools.partial(
        pl.kernel,
        out_type=jax.ShapeDtypeStruct((NW, PER_W), jnp.float32),
        mesh=mesh,
        compiler_params=pltpu.CompilerParams(use_tc_tiling_on_sc=False,
                                             needs_layout_passes=False),
        scratch_types=[
            pltpu.VMEM((L,), jnp.float32),
            pltpu.VMEM((RW,), jnp.int32),
            pltpu.VMEM((PER_W,), jnp.int32),
            pltpu.VMEM((PER_W,), jnp.int32),
            pltpu.VMEM((PER_W, S * (K + 1)), jnp.int32),
            pltpu.VMEM((PER_W, S * (K + 1)), jnp.int32),
            pltpu.VMEM((PER_W, RW), jnp.int32),
            pltpu.VMEM((PER_W, RW), jnp.int32),
            pltpu.VMEM((2 * RW, D), jnp.float32),
            pltpu.VMEM((2 * RW, D), jnp.float32),
            pltpu.VMEM((PER_W,), jnp.float32),
            pltpu.SemaphoreType.DMA,
            pltpu.SemaphoreType.DMA,
        ],
    )
    def body(emb_hbm, w_hbm, spos_hbm, uind_hbm, iind_hbm, uw_hbm, iw_hbm,
             out_hbm, w_v, spos_v, uind_v, iind_v, rawu_v, rawi_v,
             idxu_v, idxi_v, rows_a, rows_b, out_v, sem_a, sem_b):
        wid = lax.axis_index("s") * NC + lax.axis_index("c")
        pltpu.sync_copy(w_hbm, w_v)
        pltpu.sync_copy(spos_hbm, spos_v)
        pltpu.sync_copy(uind_hbm.at[wid], uind_v)
        pltpu.sync_copy(iind_hbm.at[wid], iind_v)
        pltpu.sync_copy(uw_hbm.at[wid], rawu_v)
        pltpu.sync_copy(iw_hbm.at[wid], rawi_v)

        lanes = lax.iota(jnp.int32, L)
        zero = jnp.zeros((L,), jnp.float32)

        spos_c = [spos_v[pl.ds(c * L, L)] for c in range(RW // L)]

        def build(n, carry):
            row = jnp.full((L,), n, jnp.int32)
            for raw_v, ind_v, idx_v in ((rawu_v, uind_v, idxu_v),
                                        (rawi_v, iind_v, idxi_v)):
                selfv = plsc.load_gather(ind_v, [row])
                for c in range(RW // L):
                    g = plsc.load_gather(raw_v, [row, spos_c[c]])
                    if c == 0:
                        g = jnp.where(lanes == 0, selfv, g)
                    idx_v[n, pl.ds(c * L, L)] = g
            return carry

        lax.fori_loop(0, PER_W, build, 0)

        wv = w_v[...]
        e = jnp.exp(wv - jnp.max(wv))
        wn = e / jnp.full((L,), jnp.sum(e), jnp.float32)
        w0 = jnp.sum(jnp.where(lanes == 0, wn, zero))
        w1 = jnp.sum(jnp.where(lanes == 1, wn, zero)) * (1.0 / S)
        w2 = jnp.sum(jnp.where(lanes == 2, wn, zero)) * (1.0 / S)
        w3 = jnp.sum(jnp.where(lanes == 3, wn, zero)) * (1.0 / S)
        tail_mask = lanes >= 8

        def issue(n, rows_v, sem):
            pltpu.async_copy(emb_hbm.at[idxu_v.at[n]],
                             rows_v.at[pl.ds(0, RW)], sem)
            pltpu.async_copy(emb_hbm.at[idxi_v.at[n]],
                             rows_v.at[pl.ds(RW, RW)], sem)

        def drain(rows_v, sem):
            pltpu.make_async_copy(emb_hbm.at[pl.ds(0, 2 * RW)], rows_v,
                                  sem).wait()

        def side_repr(rows_v, base):
            e0 = [rows_v[base, pl.ds(col, L)] for col in COLS]

            def group(first_row):
                def gbody(r, accs):
                    row = first_row + r
                    return tuple(acc + rows_v[row, pl.ds(col, L)]
                                 for acc, col in zip(accs, COLS))
                init = tuple(zero for _ in COLS)
                return lax.fori_loop(0, S, gbody, init)

            g1 = group(base + 1)
            g2 = group(base + 1 + S)
            g3 = group(base + 1 + 2 * S)
            return [w0 * a + w1 * b + w2 * c + w3 * d
                    for a, b, c, d in zip(e0, g1, g2, g3)]

        def compute(n, rows_v):
            u = side_repr(rows_v, 0)
            v = side_repr(rows_v, RW)
            p = zero
            for c in range(12):
                p = p + u[c] * v[c]
            tail = u[12] * v[12]
            p = p + jnp.where(tail_mask, tail, zero)
            dot = jnp.sum(p)
            plsc.store_scatter(out_v, [jnp.full((L,), n, jnp.int32)],
                               jnp.full((L,), dot, jnp.float32),
                               mask=lanes == 0)

        issue(0, rows_a, sem_a)
        issue(1, rows_b, sem_b)

        def grp(g, carry):
            for n, rows_v, sem in ((2 * g, rows_a, sem_a),
                                   (2 * g + 1, rows_b, sem_b)):
                drain(rows_v, sem)
                compute(n, rows_v)

                @pl.when(n + 2 < PER_W)
                def _():
                    issue(n + 2, rows_v, sem)
            return carry

        lax.fori_loop(0, PER_W // 2, grp, 0)
        pltpu.sync_copy(out_v, out_hbm.at[wid])

    return body(emb, wpad, spos, uind, iind, uwalks, iwalks)


def kernel(node_embeddings, user_weights, item_weights,
           user_indices, item_indices, user_walks, item_walks):
    del item_weights
    wpad = jnp.pad(user_weights, (0, L - user_weights.shape[0]),
                   constant_values=-jnp.inf)
    spos = jnp.asarray(_SPOS)
    uind = user_indices.astype(jnp.int32).reshape(NW, PER_W)
    iind = item_indices.astype(jnp.int32).reshape(NW, PER_W)
    uwalks = user_walks.astype(jnp.int32).reshape(NW, PER_W, S * (K + 1))
    iwalks = item_walks.astype(jnp.int32).reshape(NW, PER_W, S * (K + 1))
    out = _sc_scores(node_embeddings, wpad, spos, uind, iind, uwalks, iwalks)
    return out.reshape(B)


# speedup vs baseline: 1.8032x; 1.8032x over previous
import functools

import numpy as np

import jax
import jax.numpy as jnp
from jax import lax
from jax.experimental import pallas as pl
from jax.experimental.pallas import tpu as pltpu
from jax.experimental.pallas import tpu_sc as plsc

D = 200
DLO = 128
DHI = D - DLO
S = 25
K = 3
RW = 80
B = 4096
NC, NS, L = 2, 16, 16
NW = NC * NS
PER_W = B // NW
NCH_LO = DLO // (2 * L)
NCH_HI = -(-DHI // (2 * L))

_SPOS = np.zeros(RW, np.int32)
for _j in range(1, 1 + S * K):
    _k = 1 + (_j - 1) // S
    _s = (_j - 1) % S
    _SPOS[_j] = _s * (K + 1) + _k
for _j in range(1 + S * K, RW):
    _SPOS[_j] = (_j - 1 - S * K) * (K + 1)


def _sc_scores(tlo, thi, wpad, spos, uind, iind, uwalks, iwalks):
    mesh = plsc.VectorSubcoreMesh(core_axis_name="c", subcore_axis_name="s")

    @functools.partial(
        pl.kernel,
        out_type=jax.ShapeDtypeStruct((NW, PER_W), jnp.float32),
        mesh=mesh,
        compiler_params=pltpu.CompilerParams(use_tc_tiling_on_sc=False,
                                             needs_layout_passes=False),
        scratch_types=[
            pltpu.VMEM((L,), jnp.float32),
            pltpu.VMEM((RW,), jnp.int32),
            pltpu.VMEM((PER_W,), jnp.int32),
            pltpu.VMEM((PER_W,), jnp.int32),
            pltpu.VMEM((PER_W, S * (K + 1)), jnp.int32),
            pltpu.VMEM((PER_W, S * (K + 1)), jnp.int32),
            pltpu.VMEM((PER_W, RW), jnp.int32),
            pltpu.VMEM((PER_W, RW), jnp.int32),
            pltpu.VMEM((4 * RW, DLO), jnp.bfloat16),
            pltpu.VMEM((4 * RW, DLO), jnp.bfloat16),
            pltpu.VMEM((PER_W,), jnp.float32),
            pltpu.SemaphoreType.DMA,
            pltpu.SemaphoreType.DMA,
        ],
    )
    def body(tlo_hbm, thi_hbm, w_hbm, spos_hbm, uind_hbm, iind_hbm, uw_hbm,
             iw_hbm, out_hbm, w_v, spos_v, uind_v, iind_v, rawu_v, rawi_v,
             idxu_v, idxi_v, rows_a, rows_b, out_v, sem_a, sem_b):
        wid = lax.axis_index("s") * NC + lax.axis_index("c")
        pltpu.sync_copy(w_hbm, w_v)
        pltpu.sync_copy(spos_hbm, spos_v)
        pltpu.sync_copy(uind_hbm.at[wid], uind_v)
        pltpu.sync_copy(iind_hbm.at[wid], iind_v)
        pltpu.sync_copy(uw_hbm.at[wid], rawu_v)
        pltpu.sync_copy(iw_hbm.at[wid], rawi_v)

        lanes = lax.iota(jnp.int32, L)
        zero = jnp.zeros((L,), jnp.float32)

        spos_c = [spos_v[pl.ds(c * L, L)] for c in range(RW // L)]

        def build(n, carry):
            row = jnp.full((L,), n, jnp.int32)
            for raw_v, ind_v, idx_v in ((rawu_v, uind_v, idxu_v),
                                        (rawi_v, iind_v, idxi_v)):
                selfv = plsc.load_gather(ind_v, [row])
                for c in range(RW // L):
                    g = plsc.load_gather(raw_v, [row, spos_c[c]])
                    if c == 0:
                        g = jnp.where(lanes == 0, selfv, g)
                    idx_v[n, pl.ds(c * L, L)] = g
            return carry

        lax.fori_loop(0, PER_W, build, 0)

        wv = w_v[...]
        e = jnp.exp(wv - jnp.max(wv))
        wn = e / jnp.full((L,), jnp.sum(e), jnp.float32)
        w0 = jnp.sum(jnp.where(lanes == 0, wn, zero))
        w1 = jnp.sum(jnp.where(lanes == 1, wn, zero)) * (1.0 / S)
        w2 = jnp.sum(jnp.where(lanes == 2, wn, zero)) * (1.0 / S)
        w3 = jnp.sum(jnp.where(lanes == 3, wn, zero)) * (1.0 / S)

        def issue(n, rows_v, sem):
            pltpu.async_copy(tlo_hbm.at[idxu_v.at[n]],
                             rows_v.at[pl.ds(0, RW)], sem)
            pltpu.async_copy(thi_hbm.at[idxu_v.at[n]],
                             rows_v.at[pl.ds(RW, RW)], sem)
            pltpu.async_copy(tlo_hbm.at[idxi_v.at[n]],
                             rows_v.at[pl.ds(2 * RW, RW)], sem)
            pltpu.async_copy(thi_hbm.at[idxi_v.at[n]],
                             rows_v.at[pl.ds(3 * RW, RW)], sem)

        def drain(rows_v, sem):
            pltpu.make_async_copy(tlo_hbm.at[pl.ds(0, 4 * RW)], rows_v,
                                  sem).wait()

        nch = (NCH_LO, NCH_HI)

        def row_chunks(rows_v, base, j):
            out = []
            for t in range(2):
                for c in range(nch[t]):
                    bv = rows_v[base + t * RW + j, pl.ds(c * 2 * L, 2 * L)]
                    out.append(plsc.bitcast(bv, jnp.int32))
            return out

        def unpack_acc(accs, chunks, scale=None):
            res = list(accs)
            for i, v in enumerate(chunks):
                eo = (lax.bitcast_convert_type(lax.shift_left(v, 16),
                                               jnp.float32),
                      lax.bitcast_convert_type(
                          jnp.bitwise_and(v, jnp.int32(-65536)), jnp.float32))
                for h in range(2):
                    x = eo[h]
                    if scale is not None:
                        x = scale * x
                    res[2 * i + h] = res[2 * i + h] + x
            return res

        NACC = 2 * (NCH_LO + NCH_HI)

        def side_repr(rows_v, base):
            e0 = unpack_acc([zero] * NACC, row_chunks(rows_v, base, 0))

            def group(first):
                def gbody(r, accs):
                    return tuple(unpack_acc(accs,
                                            row_chunks(rows_v, base,
                                                       first + r)))
                return lax.fori_loop(0, S, gbody, tuple([zero] * NACC))

            g1 = group(1)
            g2 = group(1 + S)
            g3 = group(1 + 2 * S)
            return [w0 * a + w1 * b + w2 * c + w3 * d
                    for a, b, c, d in zip(e0, g1, g2, g3)]

        def compute(n, rows_v):
            u = side_repr(rows_v, 0)
            v = side_repr(rows_v, 2 * RW)
            p = u[0] * v[0]
            for c in range(1, NACC):
                p = p + u[c] * v[c]
            dot = jnp.sum(p)
            plsc.store_scatter(out_v, [jnp.full((L,), n, jnp.int32)],
                               jnp.full((L,), dot, jnp.float32),
                               mask=lanes == 0)

        issue(0, rows_a, sem_a)
        issue(1, rows_b, sem_b)

        def grp(g, carry):
            for n, rows_v, sem in ((2 * g, rows_a, sem_a),
                                   (2 * g + 1, rows_b, sem_b)):
                drain(rows_v, sem)
                compute(n, rows_v)

                @pl.when(n + 2 < PER_W)
                def _():
                    issue(n + 2, rows_v, sem)
            return carry

        lax.fori_loop(0, PER_W // 2, grp, 0)
        pltpu.sync_copy(out_v, out_hbm.at[wid])

    return body(tlo, thi, wpad, spos, uind, iind, uwalks, iwalks)


def kernel(node_embeddings, user_weights, item_weights,
           user_indices, item_indices, user_walks, item_walks):
    del item_weights
    wpad = jnp.pad(user_weights, (0, L - user_weights.shape[0]),
                   constant_values=-jnp.inf)
    spos = jnp.asarray(_SPOS)
    sel = np.zeros((D, 2 * DLO), np.float32)
    sel[np.arange(D), np.arange(D)] = 1.0
    sel_lo = jnp.asarray(sel[:, :DLO])
    sel_hi = jnp.asarray(sel[:, DLO:])
    dn = (((1,), (0,)), ((), ()))
    tlo = lax.dot_general(node_embeddings, sel_lo, dimension_numbers=dn,
                          preferred_element_type=jnp.bfloat16)
    thi = lax.dot_general(node_embeddings, sel_hi, dimension_numbers=dn,
                          preferred_element_type=jnp.bfloat16)
    uind = user_indices.astype(jnp.int32).reshape(NW, PER_W)
    iind = item_indices.astype(jnp.int32).reshape(NW, PER_W)
    uwalks = user_walks.astype(jnp.int32).reshape(NW, PER_W, S * (K + 1))
    iwalks = item_walks.astype(jnp.int32).reshape(NW, PER_W, S * (K + 1))
    out = _sc_scores(tlo, thi, wpad, spos, uind, iind, uwalks, iwalks)
    return out.reshape(B)
